# baseline (speedup 1.0000x reference)
import functools

import jax
import jax.numpy as jnp
from jax import lax
from jax.experimental import pallas as pl
from jax.experimental.pallas import tpu as pltpu
from jax.experimental.pallas import tpu_sc as plsc

NC = 2
NS = 16
NW = NC * NS
CHUNK = 128
NBUF = 1
IB = 10
LANES = 16


def _sc_agg_builder(n_nodes, d, nblocks, with_deg):
    chunks = nblocks * IB
    n_pad = ((n_nodes + 1 + NS * CHUNK - 1) // (NS * CHUNK)) * (NS * CHUNK)
    rows_per_tile = n_pad // NS
    zrows = rows_per_tile // CHUNK

    mesh = plsc.VectorSubcoreMesh(core_axis_name="c", subcore_axis_name="s",
                                  num_cores=NC, num_subcores=NS)

    out_type = [jax.ShapeDtypeStruct((NC, n_pad, d), jnp.float32)]
    scratch = (
        [pltpu.VMEM_SHARED((n_pad, d), jnp.float32)]
        + [pltpu.VMEM((IB, CHUNK), jnp.int32)] * 4
        + [pltpu.VMEM((CHUNK, d), jnp.float32)] * NBUF
        + [pltpu.SemaphoreType.DMA] * NBUF
    )
    if with_deg:
        out_type.append(jax.ShapeDtypeStruct((NC, n_pad), jnp.float32))
        scratch += [
            pltpu.VMEM_SHARED((n_pad,), jnp.float32),
            pltpu.VMEM((CHUNK,), jnp.float32),
            pltpu.VMEM((rows_per_tile,), jnp.float32),
        ]

    def body(table_hbm, src_hbm, dst_hbm, agg_hbm, *rest):
        if with_deg:
            deg_hbm, rest = rest[0], rest[1:]
        agg_sh = rest[0]
        src_blk = rest[1:3]
        dst_blk = rest[3:5]
        rows = rest[5:5 + NBUF]
        sems = rest[5 + NBUF:5 + 2 * NBUF]
        if with_deg:
            deg_sh, ones_v, zslab_v = rest[5 + 2 * NBUF:]
        cid = lax.axis_index("c")
        sid = lax.axis_index("s")
        wid = cid * NS + sid

        zeros16 = jnp.zeros((LANES,), jnp.float32)
        ones16 = jnp.ones((LANES,), jnp.float32)

        def zb_loop(i, carry):
            for k in range(d // LANES):
                rows[0][i, pl.ds(k * LANES, LANES)] = zeros16
            return carry
        lax.fori_loop(0, CHUNK, zb_loop, 0)

        rbase = sid * rows_per_tile
        if with_deg:
            def zd_loop(i, carry):
                zslab_v[pl.ds(i * LANES, LANES)] = zeros16
                ones_v[pl.ds((i % (CHUNK // LANES)) * LANES, LANES)] = ones16
                return carry
            lax.fori_loop(0, rows_per_tile // LANES, zd_loop, 0)
            pltpu.sync_copy(zslab_v, deg_sh.at[pl.ds(rbase, rows_per_tile)])

        for b in range(zrows):
            pltpu.sync_copy(rows[0], agg_sh.at[pl.ds(rbase + b * CHUNK, CHUNK)])

        plsc.subcore_barrier()

        pltpu.sync_copy(src_hbm.at[wid, 0], src_blk[0])
        pltpu.sync_copy(dst_hbm.at[wid, 0], dst_blk[0])
        if nblocks > 1:
            pltpu.sync_copy(src_hbm.at[wid, 1], src_blk[1])
            pltpu.sync_copy(dst_hbm.at[wid, 1], dst_blk[1])
        if NBUF > 1:
            for b in range(NBUF):
                pltpu.async_copy(table_hbm.at[src_blk[0].at[b]],
                                 rows[b], sems[b])

        for j in range(chunks):
            blk, row = divmod(j, IB)
            if row == 0 and blk >= 1 and blk + 1 < nblocks:
                pltpu.sync_copy(src_hbm.at[wid, blk + 1],
                                src_blk[(blk + 1) % 2])
                pltpu.sync_copy(dst_hbm.at[wid, blk + 1],
                                dst_blk[(blk + 1) % 2])
            rb = j % NBUF
            if NBUF == 1:
                pltpu.async_copy(table_hbm.at[src_blk[blk % 2].at[row]],
                                 rows[0], sems[0]).wait()
            else:
                pltpu.make_async_copy(table_hbm.at[pl.ds(0, CHUNK)],
                                      rows[rb], sems[rb]).wait()
            dref = dst_blk[blk % 2].at[row]
            pltpu.sync_copy(rows[rb], agg_sh.at[dref], add=True)
            if with_deg:
                pltpu.sync_copy(ones_v, deg_sh.at[dref], add=True)
            nj = j + NBUF
            if NBUF > 1 and nj < chunks:
                nblk, nrow = divmod(nj, IB)
                pltpu.async_copy(table_hbm.at[src_blk[nblk % 2].at[nrow]],
                                 rows[rb], sems[rb])

        plsc.subcore_barrier()

        pltpu.sync_copy(agg_sh.at[pl.ds(rbase, rows_per_tile)],
                        agg_hbm.at[cid, pl.ds(rbase, rows_per_tile)])
        if with_deg:
            pltpu.sync_copy(deg_sh.at[pl.ds(rbase, rows_per_tile)],
                            deg_hbm.at[cid, pl.ds(rbase, rows_per_tile)])

    kern = pl.kernel(
        body,
        out_type=out_type if with_deg else out_type[0],
        mesh=mesh,
        scratch_types=scratch,
        compiler_params=pltpu.CompilerParams(needs_layout_passes=False),
    )
    return kern, n_pad


def _tc_self_builder(n, d, n_pad, block_rows):
    grid = n_pad // block_rows

    def body(x_ref, ws_ref, b_ref, o_ref):
        o_ref[...] = (jnp.dot(x_ref[...], ws_ref[...],
                              preferred_element_type=jnp.float32)
                      + b_ref[...])

    return pl.pallas_call(
        body,
        grid=(grid,),
        in_specs=[
            pl.BlockSpec((block_rows, d), lambda i: (i, 0)),
            pl.BlockSpec((d, d), lambda i: (0, 0)),
            pl.BlockSpec((1, d), lambda i: (0, 0)),
        ],
        out_specs=pl.BlockSpec((block_rows, d), lambda i: (i, 0)),
        out_shape=jax.ShapeDtypeStruct((n, d), jnp.float32),
    )


def _tc_comb_builder(n, d, n_pad, relu, block_rows):
    grid = n_pad // block_rows

    def body(ys_ref, agg_ref, deg_ref, wn_ref, o_ref):
        agg = agg_ref[0] + agg_ref[1]
        deg = deg_ref[0] + deg_ref[1]
        inv = 1.0 / jnp.maximum(deg, 1.0)
        hn = agg * inv[:, None]
        y = ys_ref[...] + jnp.dot(hn, wn_ref[...],
                                  preferred_element_type=jnp.float32)
        if relu:
            y = jnp.maximum(y, 0.0)
        o_ref[...] = y

    return pl.pallas_call(
        body,
        grid=(grid,),
        in_specs=[
            pl.BlockSpec((block_rows, d), lambda i: (i, 0)),
            pl.BlockSpec((NC, block_rows, d), lambda i: (0, i, 0)),
            pl.BlockSpec((NC, block_rows), lambda i: (0, i)),
            pl.BlockSpec((d, d), lambda i: (0, 0)),
        ],
        out_specs=pl.BlockSpec((block_rows, d), lambda i: (i, 0)),
        out_shape=jax.ShapeDtypeStruct((n, d), jnp.float32),
    )


@functools.cache
def _build(n_nodes, d, n_edges):
    chunks = -(-n_edges // (NW * CHUNK))
    nblocks = -(-chunks // IB)
    e_pad = NW * nblocks * IB * CHUNK
    sc1, n_pad = _sc_agg_builder(n_nodes, d, nblocks, with_deg=True)
    sc2, _ = _sc_agg_builder(n_nodes, d, nblocks, with_deg=False)
    tcs = _tc_self_builder(n_nodes, d, n_pad, block_rows=1024)
    tc1 = _tc_comb_builder(n_nodes, d, n_pad, relu=True, block_rows=1024)
    tc2 = _tc_comb_builder(n_nodes, d, n_pad, relu=False, block_rows=1024)
    return sc1, sc2, tcs, tc1, tc2, e_pad, nblocks


def kernel(x, edge_index, W1_self, W1_neigh, b1, W2_self, W2_neigh, b2):
    n, d = x.shape
    e = edge_index.shape[1]
    sc1, sc2, tcs, tc1, tc2, e_pad, nblocks = _build(n, d, e)

    src = edge_index[0].astype(jnp.int32)
    dst = edge_index[1].astype(jnp.int32)
    pad = e_pad - e
    n_pad = ((n + 1 + NS * CHUNK - 1) // (NS * CHUNK)) * (NS * CHUNK)
    spare = n_pad - n
    slots = e_pad // NW
    if e % NW == 0:
        ppt = slots - e // NW
        src = src.reshape(NW, e // NW)
        dst = dst.reshape(NW, e // NW)
        if ppt:
            src = jnp.concatenate(
                [src, jnp.zeros((NW, ppt), jnp.int32)], axis=1)
            pad_dst = (n + (jnp.arange(ppt, dtype=jnp.int32)[None, :]
                            + 15 * jnp.arange(NW, dtype=jnp.int32)[:, None])
                       % spare)
            dst = jnp.concatenate([dst, pad_dst], axis=1)
    else:
        if pad:
            src = jnp.concatenate([src, jnp.zeros((pad,), jnp.int32)])
            pad_dst = n + jnp.arange(pad, dtype=jnp.int32) % spare
            dst = jnp.concatenate([dst, pad_dst])
        src = src.reshape(-1, NW).T
        dst = dst.reshape(-1, NW).T
    src = src.reshape(NW, nblocks, IB, CHUNK)
    dst = dst.reshape(NW, nblocks, IB, CHUNK)

    b1r = b1.reshape(1, d)
    b2r = b2.reshape(1, d)
    agg1, deg = sc1(x, src, dst)
    y1 = tcs(x, W1_self, b1r)
    h = tc1(y1, agg1, deg, W1_neigh)
    agg2 = sc2(h, src, dst)
    y2 = tcs(h, W2_self, b2r)
    out = tc2(y2, agg2, deg, W2_neigh)
    return out


# speedup vs baseline: 1.6063x; 1.6063x over previous
import functools

import jax
import jax.numpy as jnp
from jax import lax
from jax.experimental import pallas as pl
from jax.experimental.pallas import tpu as pltpu
from jax.experimental.pallas import tpu_sc as plsc

NC = 2
NS = 16
NW = NC * NS
CHUNK = 128
NBUF = 2
LANES = 16


def _sc_agg_builder(n_nodes, d, chunks, with_deg):
    n_pad = ((n_nodes + 1 + NS * CHUNK - 1) // (NS * CHUNK)) * (NS * CHUNK)
    rows_per_tile = n_pad // NS
    zrows = rows_per_tile // CHUNK

    mesh = plsc.VectorSubcoreMesh(core_axis_name="c", subcore_axis_name="s",
                                  num_cores=NC, num_subcores=NS)

    out_type = [jax.ShapeDtypeStruct((NC, n_pad, d), jnp.float32)]
    scratch = (
        [pltpu.VMEM_SHARED((n_pad, d), jnp.float32)]
        + [pltpu.VMEM((CHUNK,), jnp.int32)] * (2 * NBUF)
        + [pltpu.VMEM((CHUNK, d), jnp.float32)] * NBUF
        + [pltpu.SemaphoreType.DMA] * NBUF
    )
    if with_deg:
        out_type.append(jax.ShapeDtypeStruct((NC, n_pad), jnp.float32))
        scratch += [
            pltpu.VMEM_SHARED((n_pad,), jnp.float32),
            pltpu.VMEM((CHUNK,), jnp.float32),
            pltpu.VMEM((rows_per_tile,), jnp.float32),
        ]

    def body(table_hbm, src_hbm, dst_hbm, agg_hbm, *rest):
        if with_deg:
            deg_hbm, rest = rest[0], rest[1:]
        agg_sh = rest[0]
        src_b = rest[1:1 + NBUF]
        dst_b = rest[1 + NBUF:1 + 2 * NBUF]
        rows = rest[1 + 2 * NBUF:1 + 3 * NBUF]
        sems = rest[1 + 3 * NBUF:1 + 4 * NBUF]
        if with_deg:
            deg_sh, ones_v, zslab_v = rest[1 + 4 * NBUF:]
        cid = lax.axis_index("c")
        sid = lax.axis_index("s")
        wid = cid * NS + sid

        zeros16 = jnp.zeros((LANES,), jnp.float32)
        ones16 = jnp.ones((LANES,), jnp.float32)

        def zb_loop(i, carry):
            for k in range(d // LANES):
                rows[0][i, pl.ds(k * LANES, LANES)] = zeros16
            return carry
        lax.fori_loop(0, CHUNK, zb_loop, 0)

        rbase = sid * rows_per_tile
        if with_deg:
            def zd_loop(i, carry):
                zslab_v[pl.ds(i * LANES, LANES)] = zeros16
                ones_v[pl.ds((i % (CHUNK // LANES)) * LANES, LANES)] = ones16
                return carry
            lax.fori_loop(0, rows_per_tile // LANES, zd_loop, 0)
            pltpu.sync_copy(zslab_v, deg_sh.at[pl.ds(rbase, rows_per_tile)])

        for b in range(zrows):
            pltpu.sync_copy(rows[0], agg_sh.at[pl.ds(rbase + b * CHUNK, CHUNK)])

        plsc.subcore_barrier()

        for b in range(NBUF):
            pltpu.sync_copy(src_hbm.at[wid, pl.ds(b * CHUNK, CHUNK)], src_b[b])
            pltpu.sync_copy(dst_hbm.at[wid, pl.ds(b * CHUNK, CHUNK)], dst_b[b])
            pltpu.async_copy(table_hbm.at[src_b[b]], rows[b], sems[b])

        for j in range(chunks):
            rb = j % NBUF
            pltpu.make_async_copy(table_hbm.at[pl.ds(0, CHUNK)],
                                  rows[rb], sems[rb]).wait()
            pltpu.sync_copy(rows[rb], agg_sh.at[dst_b[rb]], add=True)
            if with_deg:
                pltpu.sync_copy(ones_v, deg_sh.at[dst_b[rb]], add=True)
            nj = j + NBUF
            if nj < chunks:
                pltpu.sync_copy(src_hbm.at[wid, pl.ds(nj * CHUNK, CHUNK)],
                                src_b[rb])
                pltpu.sync_copy(dst_hbm.at[wid, pl.ds(nj * CHUNK, CHUNK)],
                                dst_b[rb])
                pltpu.async_copy(table_hbm.at[src_b[rb]], rows[rb], sems[rb])

        plsc.subcore_barrier()

        pltpu.sync_copy(agg_sh.at[pl.ds(rbase, rows_per_tile)],
                        agg_hbm.at[cid, pl.ds(rbase, rows_per_tile)])
        if with_deg:
            pltpu.sync_copy(deg_sh.at[pl.ds(rbase, rows_per_tile)],
                            deg_hbm.at[cid, pl.ds(rbase, rows_per_tile)])

    kern = pl.kernel(
        body,
        out_type=out_type if with_deg else out_type[0],
        mesh=mesh,
        scratch_types=scratch,
        compiler_params=pltpu.CompilerParams(needs_layout_passes=False),
    )
    return kern, n_pad


def _tc_self_builder(n, d, n_pad, block_rows):
    grid = n_pad // block_rows

    def body(x_ref, ws_ref, b_ref, o_ref):
        o_ref[...] = (jnp.dot(x_ref[...], ws_ref[...],
                              preferred_element_type=jnp.float32)
                      + b_ref[...])

    return pl.pallas_call(
        body,
        grid=(grid,),
        in_specs=[
            pl.BlockSpec((block_rows, d), lambda i: (i, 0)),
            pl.BlockSpec((d, d), lambda i: (0, 0)),
            pl.BlockSpec((1, d), lambda i: (0, 0)),
        ],
        out_specs=pl.BlockSpec((block_rows, d), lambda i: (i, 0)),
        out_shape=jax.ShapeDtypeStruct((n, d), jnp.float32),
    )


def _tc_comb_builder(n, d, n_pad, relu, block_rows):
    grid = n_pad // block_rows

    def body(ys_ref, agg_ref, deg_ref, wn_ref, o_ref):
        agg = agg_ref[0] + agg_ref[1]
        deg = deg_ref[0] + deg_ref[1]
        inv = 1.0 / jnp.maximum(deg, 1.0)
        hn = agg * inv[:, None]
        y = ys_ref[...] + jnp.dot(hn, wn_ref[...],
                                  preferred_element_type=jnp.float32)
        if relu:
            y = jnp.maximum(y, 0.0)
        o_ref[...] = y

    return pl.pallas_call(
        body,
        grid=(grid,),
        in_specs=[
            pl.BlockSpec((block_rows, d), lambda i: (i, 0)),
            pl.BlockSpec((NC, block_rows, d), lambda i: (0, i, 0)),
            pl.BlockSpec((NC, block_rows), lambda i: (0, i)),
            pl.BlockSpec((d, d), lambda i: (0, 0)),
        ],
        out_specs=pl.BlockSpec((block_rows, d), lambda i: (i, 0)),
        out_shape=jax.ShapeDtypeStruct((n, d), jnp.float32),
    )


@functools.cache
def _build(n_nodes, d, n_edges):
    chunks = -(-n_edges // (NW * CHUNK))
    chunks = max(chunks, NBUF)
    e_pad = NW * chunks * CHUNK
    sc1, n_pad = _sc_agg_builder(n_nodes, d, chunks, with_deg=True)
    sc2, _ = _sc_agg_builder(n_nodes, d, chunks, with_deg=False)
    tcs = _tc_self_builder(n_nodes, d, n_pad, block_rows=1024)
    tc1 = _tc_comb_builder(n_nodes, d, n_pad, relu=True, block_rows=1024)
    tc2 = _tc_comb_builder(n_nodes, d, n_pad, relu=False, block_rows=1024)
    return sc1, sc2, tcs, tc1, tc2, e_pad


def kernel(x, edge_index, W1_self, W1_neigh, b1, W2_self, W2_neigh, b2):
    n, d = x.shape
    e = edge_index.shape[1]
    sc1, sc2, tcs, tc1, tc2, e_pad = _build(n, d, e)

    src = edge_index[0].astype(jnp.int32)
    dst = edge_index[1].astype(jnp.int32)
    pad = e_pad - e
    n_pad = ((n + 1 + NS * CHUNK - 1) // (NS * CHUNK)) * (NS * CHUNK)
    spare = n_pad - n
    slots = e_pad // NW
    if e % NW == 0:
        ppt = slots - e // NW
        src = src.reshape(NW, e // NW)
        dst = dst.reshape(NW, e // NW)
        if ppt:
            src = jnp.concatenate(
                [src, jnp.zeros((NW, ppt), jnp.int32)], axis=1)
            pad_dst = (n + (jnp.arange(ppt, dtype=jnp.int32)[None, :]
                            + 15 * jnp.arange(NW, dtype=jnp.int32)[:, None])
                       % spare)
            dst = jnp.concatenate([dst, pad_dst], axis=1)
    else:
        if pad:
            src = jnp.concatenate([src, jnp.zeros((pad,), jnp.int32)])
            pad_dst = n + jnp.arange(pad, dtype=jnp.int32) % spare
            dst = jnp.concatenate([dst, pad_dst])
        src = src.reshape(-1, NW).T
        dst = dst.reshape(-1, NW).T
    src = src.reshape(NW, slots)
    dst = dst.reshape(NW, slots)

    b1r = b1.reshape(1, d)
    b2r = b2.reshape(1, d)
    agg1, deg = sc1(x, src, dst)
    y1 = tcs(x, W1_self, b1r)
    h = tc1(y1, agg1, deg, W1_neigh)
    agg2 = sc2(h, src, dst)
    y2 = tcs(h, W2_self, b2r)
    out = tc2(y2, agg2, deg, W2_neigh)
    return out


# speedup vs baseline: 1.7006x; 1.0587x over previous
import functools

import jax
import jax.numpy as jnp
from jax import lax
from jax.experimental import pallas as pl
from jax.experimental.pallas import tpu as pltpu
from jax.experimental.pallas import tpu_sc as plsc

NC = 2
NS = 16
NW = NC * NS
CHUNK = 64
NBUF = 4
LANES = 16


def _sc_agg_builder(n_nodes, d, chunks, with_deg):
    n_pad = ((n_nodes + 1 + NS * CHUNK - 1) // (NS * CHUNK)) * (NS * CHUNK)
    rows_per_tile = n_pad // NS
    zrows = rows_per_tile // CHUNK

    mesh = plsc.VectorSubcoreMesh(core_axis_name="c", subcore_axis_name="s",
                                  num_cores=NC, num_subcores=NS)

    out_type = [jax.ShapeDtypeStruct((NC, n_pad, d), jnp.float32)]
    scratch = (
        [pltpu.VMEM_SHARED((n_pad, d), jnp.float32)]
        + [pltpu.VMEM((CHUNK,), jnp.int32)] * (2 * NBUF)
        + [pltpu.VMEM((CHUNK, d), jnp.float32)] * NBUF
        + [pltpu.SemaphoreType.DMA] * NBUF
    )
    if with_deg:
        out_type.append(jax.ShapeDtypeStruct((NC, n_pad), jnp.float32))
        scratch += [
            pltpu.VMEM_SHARED((n_pad,), jnp.float32),
            pltpu.VMEM((CHUNK,), jnp.float32),
            pltpu.VMEM((rows_per_tile,), jnp.float32),
        ]

    def body(table_hbm, src_hbm, dst_hbm, agg_hbm, *rest):
        if with_deg:
            deg_hbm, rest = rest[0], rest[1:]
        agg_sh = rest[0]
        src_b = rest[1:1 + NBUF]
        dst_b = rest[1 + NBUF:1 + 2 * NBUF]
        rows = rest[1 + 2 * NBUF:1 + 3 * NBUF]
        sems = rest[1 + 3 * NBUF:1 + 4 * NBUF]
        if with_deg:
            deg_sh, ones_v, zslab_v = rest[1 + 4 * NBUF:]
        cid = lax.axis_index("c")
        sid = lax.axis_index("s")
        wid = cid * NS + sid

        zeros16 = jnp.zeros((LANES,), jnp.float32)
        ones16 = jnp.ones((LANES,), jnp.float32)

        def zb_loop(i, carry):
            for k in range(d // LANES):
                rows[0][i, pl.ds(k * LANES, LANES)] = zeros16
            return carry
        lax.fori_loop(0, CHUNK, zb_loop, 0)

        rbase = sid * rows_per_tile
        if with_deg:
            def zd_loop(i, carry):
                zslab_v[pl.ds(i * LANES, LANES)] = zeros16
                ones_v[pl.ds((i % (CHUNK // LANES)) * LANES, LANES)] = ones16
                return carry
            lax.fori_loop(0, rows_per_tile // LANES, zd_loop, 0)
            pltpu.sync_copy(zslab_v, deg_sh.at[pl.ds(rbase, rows_per_tile)])

        for b in range(zrows):
            pltpu.sync_copy(rows[0], agg_sh.at[pl.ds(rbase + b * CHUNK, CHUNK)])

        plsc.subcore_barrier()

        for b in range(NBUF):
            pltpu.sync_copy(src_hbm.at[wid, pl.ds(b * CHUNK, CHUNK)], src_b[b])
            pltpu.sync_copy(dst_hbm.at[wid, pl.ds(b * CHUNK, CHUNK)], dst_b[b])
            pltpu.async_copy(table_hbm.at[src_b[b]], rows[b], sems[b])

        for j in range(chunks):
            rb = j % NBUF
            pltpu.make_async_copy(table_hbm.at[pl.ds(0, CHUNK)],
                                  rows[rb], sems[rb]).wait()
            pltpu.sync_copy(rows[rb], agg_sh.at[dst_b[rb]], add=True)
            if with_deg:
                pltpu.sync_copy(ones_v, deg_sh.at[dst_b[rb]], add=True)
            nj = j + NBUF
            if nj < chunks:
                pltpu.sync_copy(src_hbm.at[wid, pl.ds(nj * CHUNK, CHUNK)],
                                src_b[rb])
                pltpu.sync_copy(dst_hbm.at[wid, pl.ds(nj * CHUNK, CHUNK)],
                                dst_b[rb])
                pltpu.async_copy(table_hbm.at[src_b[rb]], rows[rb], sems[rb])

        plsc.subcore_barrier()

        pltpu.sync_copy(agg_sh.at[pl.ds(rbase, rows_per_tile)],
                        agg_hbm.at[cid, pl.ds(rbase, rows_per_tile)])
        if with_deg:
            pltpu.sync_copy(deg_sh.at[pl.ds(rbase, rows_per_tile)],
                            deg_hbm.at[cid, pl.ds(rbase, rows_per_tile)])

    kern = pl.kernel(
        body,
        out_type=out_type if with_deg else out_type[0],
        mesh=mesh,
        scratch_types=scratch,
        compiler_params=pltpu.CompilerParams(needs_layout_passes=False),
    )
    return kern, n_pad


def _tc_self_builder(n, d, n_pad, block_rows):
    grid = n_pad // block_rows

    def body(x_ref, ws_ref, b_ref, o_ref):
        o_ref[...] = (jnp.dot(x_ref[...], ws_ref[...],
                              preferred_element_type=jnp.float32)
                      + b_ref[...])

    return pl.pallas_call(
        body,
        grid=(grid,),
        in_specs=[
            pl.BlockSpec((block_rows, d), lambda i: (i, 0)),
            pl.BlockSpec((d, d), lambda i: (0, 0)),
            pl.BlockSpec((1, d), lambda i: (0, 0)),
        ],
        out_specs=pl.BlockSpec((block_rows, d), lambda i: (i, 0)),
        out_shape=jax.ShapeDtypeStruct((n, d), jnp.float32),
    )


def _tc_comb_builder(n, d, n_pad, relu, block_rows):
    grid = n_pad // block_rows

    def body(ys_ref, agg_ref, deg_ref, wn_ref, o_ref):
        agg = agg_ref[0] + agg_ref[1]
        deg = deg_ref[0] + deg_ref[1]
        inv = 1.0 / jnp.maximum(deg, 1.0)
        hn = agg * inv[:, None]
        y = ys_ref[...] + jnp.dot(hn, wn_ref[...],
                                  preferred_element_type=jnp.float32)
        if relu:
            y = jnp.maximum(y, 0.0)
        o_ref[...] = y

    return pl.pallas_call(
        body,
        grid=(grid,),
        in_specs=[
            pl.BlockSpec((block_rows, d), lambda i: (i, 0)),
            pl.BlockSpec((NC, block_rows, d), lambda i: (0, i, 0)),
            pl.BlockSpec((NC, block_rows), lambda i: (0, i)),
            pl.BlockSpec((d, d), lambda i: (0, 0)),
        ],
        out_specs=pl.BlockSpec((block_rows, d), lambda i: (i, 0)),
        out_shape=jax.ShapeDtypeStruct((n, d), jnp.float32),
    )


@functools.cache
def _build(n_nodes, d, n_edges):
    chunks = -(-n_edges // (NW * CHUNK))
    chunks = max(chunks, NBUF)
    e_pad = NW * chunks * CHUNK
    sc1, n_pad = _sc_agg_builder(n_nodes, d, chunks, with_deg=True)
    sc2, _ = _sc_agg_builder(n_nodes, d, chunks, with_deg=False)
    tcs = _tc_self_builder(n_nodes, d, n_pad, block_rows=1024)
    tc1 = _tc_comb_builder(n_nodes, d, n_pad, relu=True, block_rows=1024)
    tc2 = _tc_comb_builder(n_nodes, d, n_pad, relu=False, block_rows=1024)
    return sc1, sc2, tcs, tc1, tc2, e_pad


def kernel(x, edge_index, W1_self, W1_neigh, b1, W2_self, W2_neigh, b2):
    n, d = x.shape
    e = edge_index.shape[1]
    sc1, sc2, tcs, tc1, tc2, e_pad = _build(n, d, e)

    src = edge_index[0].astype(jnp.int32)
    dst = edge_index[1].astype(jnp.int32)
    pad = e_pad - e
    n_pad = ((n + 1 + NS * CHUNK - 1) // (NS * CHUNK)) * (NS * CHUNK)
    spare = n_pad - n
    slots = e_pad // NW
    if e % NW == 0:
        ppt = slots - e // NW
        src = src.reshape(NW, e // NW)
        dst = dst.reshape(NW, e // NW)
        if ppt:
            src = jnp.concatenate(
                [src, jnp.zeros((NW, ppt), jnp.int32)], axis=1)
            pad_dst = (n + (jnp.arange(ppt, dtype=jnp.int32)[None, :]
                            + 15 * jnp.arange(NW, dtype=jnp.int32)[:, None])
                       % spare)
            dst = jnp.concatenate([dst, pad_dst], axis=1)
    else:
        if pad:
            src = jnp.concatenate([src, jnp.zeros((pad,), jnp.int32)])
            pad_dst = n + jnp.arange(pad, dtype=jnp.int32) % spare
            dst = jnp.concatenate([dst, pad_dst])
        src = src.reshape(-1, NW).T
        dst = dst.reshape(-1, NW).T
    src = src.reshape(NW, slots)
    dst = dst.reshape(NW, slots)

    b1r = b1.reshape(1, d)
    b2r = b2.reshape(1, d)
    agg1, deg = sc1(x, src, dst)
    y1 = tcs(x, W1_self, b1r)
    h = tc1(y1, agg1, deg, W1_neigh)
    agg2 = sc2(h, src, dst)
    y2 = tcs(h, W2_self, b2r)
    out = tc2(y2, agg2, deg, W2_neigh)
    return out
